# baseline (speedup 1.0000x reference)
import functools

import jax
import jax.numpy as jnp
from jax import lax
from jax.experimental import pallas as pl
from jax.experimental.pallas import tpu as pltpu
from jax.experimental.pallas import tpu_sc as plsc

L = 16
NC = 2
NS = 16
NW = NC * NS
IDX_CHUNK = 128


def _make_sc_gather(total, dim, vocab, chunk, chunks_per_row):
    mod = vocab - 1
    n_gathers = chunk // IDX_CHUNK
    n_vecs = chunk // L

    @functools.partial(
        pl.kernel,
        out_type=jax.ShapeDtypeStruct((total, dim), jnp.float32),
        mesh=plsc.VectorSubcoreMesh(core_axis_name="c", subcore_axis_name="s",
                                    num_cores=NC, num_subcores=NS),
        scratch_types=[
            pltpu.VMEM((chunk + 8,), jnp.int32),
            pltpu.VMEM((n_gathers, IDX_CHUNK), jnp.int32),
            pltpu.VMEM((chunk, dim), jnp.float32),
            pltpu.VMEM((L,), jnp.float32),
            pltpu.SemaphoreType.DMA,
        ],
        compiler_params=pltpu.CompilerParams(use_tc_tiling_on_sc=False),
    )
    def body(ids_hbm, scale_hbm, table_hbm, out_hbm, buf, idx2, rows, sv_ref, sem):
        cid = lax.axis_index("c")
        sid = lax.axis_index("s")
        wid = sid * NC + cid
        base = wid * chunk
        row_pos = lax.rem(wid, chunks_per_row)
        at_row_start = row_pos == 0

        pltpu.sync_copy(scale_hbm, sv_ref)

        @pl.when(at_row_start)
        def _():
            pltpu.sync_copy(ids_hbm.at[pl.ds(base, chunk)],
                            buf.at[pl.ds(8, chunk)])

        @pl.when(jnp.logical_not(at_row_start))
        def _():
            pltpu.sync_copy(ids_hbm.at[pl.ds(base - 8, chunk + 8)], buf)

        lane = lax.iota(jnp.int32, L)
        row_pos_vec = jnp.full((L,), row_pos, jnp.int32)
        first_key = lane + row_pos_vec * jnp.int32(64)
        for k in range(n_vecs):
            cur = buf[pl.ds(8 + k * L, L)]
            prev = buf[pl.ds(7 + k * L, L)]
            h = jnp.mod((cur * jnp.int32(36313)) ^ (prev * jnp.int32(27191)),
                        jnp.int32(mod))
            if k == 0:
                h = jnp.where(first_key == 0, jnp.int32(mod), h)
            idx2[k * L // IDX_CHUNK, pl.ds((k * L) % IDX_CHUNK, L)] = h

        copies = []
        for g in range(n_gathers):
            copies.append(pltpu.async_copy(
                table_hbm.at[idx2.at[g]],
                rows.at[pl.ds(g * IDX_CHUNK, IDX_CHUNK)],
                sem,
            ))
        for cp in copies:
            cp.wait()

        sv = sv_ref[...]

        def scale_row(i, carry):
            for cc in range(dim // L):
                rows[i, pl.ds(cc * L, L)] = rows[i, pl.ds(cc * L, L)] * sv
            return carry

        lax.fori_loop(0, chunk, scale_row, 0)

        pltpu.sync_copy(rows, out_hbm.at[pl.ds(base, chunk)])

    return body


def kernel(ids, embed_weight, scale):
    b, s = ids.shape
    vocab, dim = embed_weight.shape
    total = b * s
    chunk = total // NW
    chunks_per_row = s // chunk
    ids_flat = ids.reshape(total)
    scale_vec = jnp.broadcast_to(scale.astype(jnp.float32), (L,))
    fn = _make_sc_gather(total, dim, vocab, chunk, chunks_per_row)
    out = fn(ids_flat, scale_vec, embed_weight)
    return out.reshape(b, s, dim)
